# baseline (speedup 1.0000x reference)
import functools

import jax
import jax.numpy as jnp
from jax import lax
from jax.experimental import pallas as pl
from jax.experimental.pallas import tpu as pltpu
from jax.experimental.pallas import tpu_sc as plsc

NE = 10
EP = 128
DM = 1024
NTOK = 8192
TOPK = 2
NPAIR = NTOK * TOPK
BM = 512
NWRK = 32
CHP = NPAIR // NWRK
CHT = NTOK // NWRK
BMG = 256
TMAX = NPAIR // BMG + NE
PP = TMAX * BMG
SUB = 64
NSUB = CHP // SUB
CSUB = 32
NCSUB = CHT // CSUB


def _router_body(xbf_ref, wr_ref, idx_ref, cnt_ref, eot_ref, tot_ref):
    m = pl.program_id(0)
    logits = jax.lax.dot_general(
        xbf_ref[...], wr_ref[...], (((1,), (0,)), ((), ())),
        preferred_element_type=jnp.float32)
    lane = jax.lax.broadcasted_iota(jnp.int32, (BM, EP), 1)
    neg = jnp.float32(-jnp.inf)
    logits = jnp.where(lane < NE, logits, neg)
    m1 = jnp.max(logits, axis=1, keepdims=True)
    i1 = jnp.min(jnp.where(logits == m1, lane, EP), axis=1, keepdims=True)
    l2 = jnp.where(lane == i1, neg, logits)
    m2 = jnp.max(l2, axis=1, keepdims=True)
    i2 = jnp.min(jnp.where(l2 == m2, lane, EP), axis=1, keepdims=True)
    idx_ref[...] = jnp.concatenate([i1, i2], axis=1)
    sel = (lane == i1) | (lane == i2)
    selc = sel.astype(jnp.int32)
    h0 = jnp.sum(selc[: BM // 2], axis=0, keepdims=True)
    h1 = jnp.sum(selc[BM // 2 :], axis=0, keepdims=True)
    cnt_ref[0] = jnp.concatenate([h0, h1], axis=0)

    @pl.when(m == 0)
    def _tot_init():
        tot_ref[...] = h0 + h1

    @pl.when(m > 0)
    def _tot_acc():
        tot_ref[...] += h0 + h1

    @pl.when(m == NTOK // BM - 1)
    def _eot():
        tot = tot_ref[...]
        tiles = jnp.where(lane[:1] < NE, (tot + (BMG - 1)) >> 8, 0)
        eot = jnp.zeros((1, EP), jnp.int32)
        run = jnp.zeros((1, 1), jnp.int32)
        for e in range(NE - 1):
            run = run + tiles[0:1, e:e + 1]
            eot += (lane[:1] >= run).astype(jnp.int32)
        eot_ref[...] = jnp.minimum(eot, NE - 1)


def _router(xbf, wr):
    return pl.pallas_call(
        _router_body,
        grid=(NTOK // BM,),
        in_specs=[
            pl.BlockSpec((BM, DM), lambda m: (m, 0)),
            pl.BlockSpec((DM, EP), lambda m: (0, 0)),
        ],
        out_specs=[
            pl.BlockSpec((BM, TOPK), lambda m: (m, 0)),
            pl.BlockSpec((1, 2, EP), lambda m: (m, 0, 0)),
            pl.BlockSpec((1, EP), lambda m: (0, 0)),
        ],
        out_shape=[
            jax.ShapeDtypeStruct((NTOK, TOPK), jnp.int32),
            jax.ShapeDtypeStruct((NTOK // BM, 2, EP), jnp.int32),
            jax.ShapeDtypeStruct((1, EP), jnp.int32),
        ],
        scratch_shapes=[pltpu.VMEM((1, EP), jnp.int32)],
    )(xbf, wr)


def _lane16():
    return jax.lax.iota(jnp.int32, 16)


def _dispatch_body(ef_hbm, x_hbm, cnt_hbm, xs_hbm, inv_hbm,
                   ef_v, all_v, roff_v, qa, ta, rows_v, sem):
    wid = lax.axis_index("s") * 2 + lax.axis_index("c")
    lane = _lane16()
    pltpu.sync_copy(ef_hbm.at[wid], ef_v)
    pltpu.sync_copy(cnt_hbm, all_v)

    def acc_body(w, carry):
        tot, base = carry
        row = all_v[w]
        tot = tot + row
        base = base + jnp.where(w < wid, row, 0)
        return tot, base

    tot, base_w = lax.fori_loop(
        0, NWRK, acc_body,
        (jnp.zeros((16,), jnp.int32), jnp.zeros((16,), jnp.int32)))

    tiles = (tot + (BMG - 1)) >> 8
    incl = plsc.cumsum(tiles)
    pad_start = (incl - tiles) * BMG
    roff_v[...] = pad_start + base_w

    for s in range(NSUB):
        def place_body(i, _):
            k = s * 4 + i
            v = ef_v[pl.ds(k * 16, 16)]
            base = plsc.load_gather(roff_v, [v])
            r = jnp.zeros((16,), jnp.int32)
            cnt = jnp.zeros((16,), jnp.int32)
            for e in range(NE):
                m = v == e
                mc = plsc.cumsum(m.astype(jnp.int32))
                r = jnp.where(m, mc - 1, r)
                c = plsc.all_reduce_population_count(m)
                cnt = cnt + jnp.where(lane == e, c, 0)
            q = base + r
            qa.at[s][pl.ds(i * 16, 16)] = q
            tok = (wid * CHP + k * 16 + lane) >> 1
            ta.at[s][pl.ds(i * 16, 16)] = tok
            roff_v[...] = roff_v[...] + cnt
            return 0

        lax.fori_loop(0, 4, place_body, 0)

    pltpu.sync_copy(qa, inv_hbm.at[wid])

    for s in range(NSUB):
        pltpu.async_copy(x_hbm.at[ta.at[s]], rows_v, sem).wait()
        pltpu.sync_copy(rows_v, xs_hbm.at[qa.at[s]])


def _dispatch(ef, x, counts):
    mesh = plsc.VectorSubcoreMesh(core_axis_name="c", subcore_axis_name="s")
    run = pl.kernel(
        _dispatch_body,
        mesh=mesh,
        out_type=[
            jax.ShapeDtypeStruct((PP, DM // 2), jnp.int32),
            jax.ShapeDtypeStruct((NWRK, NSUB, SUB), jnp.int32),
        ],
        scratch_types=[
            pltpu.VMEM((CHP,), jnp.int32),
            pltpu.VMEM((NWRK, 16), jnp.int32),
            pltpu.VMEM((16,), jnp.int32),
            pltpu.VMEM((NSUB, SUB), jnp.int32),
            pltpu.VMEM((NSUB, SUB), jnp.int32),
            pltpu.VMEM((SUB, DM // 2), jnp.int32),
            pltpu.SemaphoreType.DMA,
        ],
        compiler_params=pltpu.CompilerParams(needs_layout_passes=False),
    )
    return run(ef, x, counts)


def _gmm_body(e_ref, xs_ref, w_ref, b_ref, ys_ref):
    xi = xs_ref[...]
    lo = jax.lax.bitcast_convert_type(xi << 16, jnp.float32)
    hi = jax.lax.bitcast_convert_type((xi >> 16) << 16, jnp.float32)
    y = (jnp.dot(lo, w_ref[0, :, 0, :], preferred_element_type=jnp.float32)
         + jnp.dot(hi, w_ref[0, :, 1, :], preferred_element_type=jnp.float32))
    ys_ref[...] = (y + b_ref[0]) * jnp.float32(0.5)


def _gmm(e_of_tile, xs, W, b3):
    grid_spec = pltpu.PrefetchScalarGridSpec(
        num_scalar_prefetch=1,
        grid=(TMAX,),
        in_specs=[
            pl.BlockSpec((BMG, DM // 2), lambda t, e_ref: (t, 0)),
            pl.BlockSpec((1, DM // 2, 2, DM), lambda t, e_ref: (e_ref[t], 0, 0, 0)),
            pl.BlockSpec((1, 1, DM), lambda t, e_ref: (e_ref[t], 0, 0)),
        ],
        out_specs=pl.BlockSpec((BMG, DM), lambda t, e_ref: (t, 0)),
    )
    return pl.pallas_call(
        _gmm_body,
        grid_spec=grid_spec,
        out_shape=jax.ShapeDtypeStruct((PP, DM), jnp.float32),
    )(e_of_tile, xs, W.reshape(NE, DM // 2, 2, DM), b3)


def _combine_body(ys_hbm, inv_hbm, out_hbm, inv_v, rows_v, ov, sem):
    wid = lax.axis_index("s") * 2 + lax.axis_index("c")
    lane = _lane16()
    pltpu.sync_copy(inv_hbm.at[wid], inv_v)

    def chunk_body(s, _):
        pltpu.async_copy(ys_hbm.at[inv_v.at[s]], rows_v, sem).wait()

        @plsc.parallel_loop(0, CSUB * (DM // 16), unroll=8)
        def add_body(i):
            j = i >> 6
            c = i & 63
            col = c * 16 + lane
            ra = jnp.zeros((16,), jnp.int32) + 2 * j
            a = plsc.load_gather(rows_v, [ra, col])
            bb = plsc.load_gather(rows_v, [ra + 1, col])
            ov[pl.ds(j * DM + c * 16, 16)] = a + bb

        pltpu.sync_copy(
            ov, out_hbm.at[pl.ds((wid * CHT + s * CSUB) * DM, CSUB * DM)])
        return 0

    lax.fori_loop(0, NCSUB, chunk_body, 0)


def _combine(ys, inv):
    mesh = plsc.VectorSubcoreMesh(core_axis_name="c", subcore_axis_name="s")
    run = pl.kernel(
        _combine_body,
        mesh=mesh,
        out_type=jax.ShapeDtypeStruct((NTOK * DM,), jnp.float32),
        scratch_types=[
            pltpu.VMEM((NCSUB, 2 * CSUB), jnp.int32),
            pltpu.VMEM((2 * CSUB, DM), jnp.float32),
            pltpu.VMEM((CSUB * DM,), jnp.float32),
            pltpu.SemaphoreType.DMA,
        ],
        compiler_params=pltpu.CompilerParams(needs_layout_passes=False),
    )
    return run(ys, inv)


@jax.jit
def kernel(x, W, b, Wr, br):
    xbf = x.astype(jnp.bfloat16)
    xpk = jax.lax.bitcast_convert_type(
        xbf.reshape(NTOK, DM // 2, 2), jnp.int32)
    wr = jnp.pad(Wr.astype(jnp.bfloat16), ((0, 0), (0, EP - NE)))
    idx, cnt, e_of_tile = _router(xbf, wr)

    counts = cnt.reshape(NWRK, EP)[:, :16]
    ef = idx.reshape(NWRK, CHP)

    xs, inv = _dispatch(ef, xpk, counts)
    ys = _gmm(e_of_tile.reshape(EP), xs, W, b[:, None, :])
    out = _combine(ys, inv)
    return out.reshape(NTOK, DM)


# speedup vs baseline: 1.4984x; 1.4984x over previous
import functools

import jax
import jax.numpy as jnp
from jax import lax
from jax.experimental import pallas as pl
from jax.experimental.pallas import tpu as pltpu
from jax.experimental.pallas import tpu_sc as plsc

NE = 10
EP = 128
DM = 1024
NTOK = 8192
TOPK = 2
NPAIR = NTOK * TOPK
BM = 512
NWRK = 32
CHP = NPAIR // NWRK
CHT = NTOK // NWRK
BMG = 256
TMAX = NPAIR // BMG + NE
PP = TMAX * BMG
SUB = 64
NSUB = CHP // SUB
CSUB = 32
NCSUB = CHT // CSUB


def _router_body(xbf_ref, wr_ref, idx_ref, cnt_ref, eot_ref, tot_ref):
    m = pl.program_id(0)
    logits = jax.lax.dot_general(
        xbf_ref[...], wr_ref[...], (((1,), (0,)), ((), ())),
        preferred_element_type=jnp.float32)
    lane = jax.lax.broadcasted_iota(jnp.int32, (BM, EP), 1)
    neg = jnp.float32(-jnp.inf)
    logits = jnp.where(lane < NE, logits, neg)
    m1 = jnp.max(logits, axis=1, keepdims=True)
    i1 = jnp.min(jnp.where(logits == m1, lane, EP), axis=1, keepdims=True)
    l2 = jnp.where(lane == i1, neg, logits)
    m2 = jnp.max(l2, axis=1, keepdims=True)
    i2 = jnp.min(jnp.where(l2 == m2, lane, EP), axis=1, keepdims=True)
    idx_ref[...] = jnp.concatenate([i1, i2], axis=1)
    sel = (lane == i1) | (lane == i2)
    selc = sel.astype(jnp.int32)
    h0 = jnp.sum(selc[: BM // 2], axis=0, keepdims=True)
    h1 = jnp.sum(selc[BM // 2 :], axis=0, keepdims=True)
    cnt_ref[0] = jnp.concatenate([h0, h1], axis=0)

    @pl.when(m == 0)
    def _tot_init():
        tot_ref[...] = h0 + h1

    @pl.when(m > 0)
    def _tot_acc():
        tot_ref[...] += h0 + h1

    @pl.when(m == NTOK // BM - 1)
    def _eot():
        tot = tot_ref[...]
        tiles = jnp.where(lane[:1] < NE, (tot + (BMG - 1)) >> 8, 0)
        eot = jnp.zeros((1, EP), jnp.int32)
        run = jnp.zeros((1, 1), jnp.int32)
        for e in range(NE - 1):
            run = run + tiles[0:1, e:e + 1]
            eot += (lane[:1] >= run).astype(jnp.int32)
        eot_ref[...] = jnp.minimum(eot, NE - 1)


def _router(xbf, wr):
    return pl.pallas_call(
        _router_body,
        grid=(NTOK // BM,),
        in_specs=[
            pl.BlockSpec((BM, DM), lambda m: (m, 0)),
            pl.BlockSpec((DM, EP), lambda m: (0, 0)),
        ],
        out_specs=[
            pl.BlockSpec((BM, TOPK), lambda m: (m, 0)),
            pl.BlockSpec((1, 2, EP), lambda m: (m, 0, 0)),
            pl.BlockSpec((1, EP), lambda m: (0, 0)),
        ],
        out_shape=[
            jax.ShapeDtypeStruct((NTOK, TOPK), jnp.int32),
            jax.ShapeDtypeStruct((NTOK // BM, 2, EP), jnp.int32),
            jax.ShapeDtypeStruct((1, EP), jnp.int32),
        ],
        scratch_shapes=[pltpu.VMEM((1, EP), jnp.int32)],
    )(xbf, wr)


def _lane16():
    return jax.lax.iota(jnp.int32, 16)


def _dispatch_body(ef_hbm, x_hbm, cnt_hbm, xs_hbm, inv_hbm,
                   ef_v, all_v, roff_v, qa, ta, rows_v, sem):
    wid = lax.axis_index("s") * 2 + lax.axis_index("c")
    lane = _lane16()
    pltpu.sync_copy(ef_hbm.at[wid], ef_v)
    pltpu.sync_copy(cnt_hbm, all_v)

    def acc_body(w, carry):
        tot, base = carry
        row = all_v[w]
        tot = tot + row
        base = base + jnp.where(w < wid, row, 0)
        return tot, base

    tot, base_w = lax.fori_loop(
        0, NWRK, acc_body,
        (jnp.zeros((16,), jnp.int32), jnp.zeros((16,), jnp.int32)))

    tiles = (tot + (BMG - 1)) >> 8
    incl = plsc.cumsum(tiles)
    pad_start = (incl - tiles) * BMG
    roff_v[...] = pad_start + base_w

    for s in range(NSUB):
        def place_body(i, _):
            k = s * 4 + i
            v = ef_v[pl.ds(k * 16, 16)]
            base = plsc.load_gather(roff_v, [v])
            r = jnp.zeros((16,), jnp.int32)
            cnt = jnp.zeros((16,), jnp.int32)
            for e in range(NE):
                m = v == e
                mc = plsc.cumsum(m.astype(jnp.int32))
                r = jnp.where(m, mc - 1, r)
                c = plsc.all_reduce_population_count(m)
                cnt = cnt + jnp.where(lane == e, c, 0)
            q = base + r
            qa.at[s][pl.ds(i * 16, 16)] = q
            tok = (wid * CHP + k * 16 + lane) >> 1
            ta.at[s][pl.ds(i * 16, 16)] = tok
            roff_v[...] = roff_v[...] + cnt
            return 0

        lax.fori_loop(0, 4, place_body, 0)

    pltpu.sync_copy(qa, inv_hbm.at[wid])

    for s in range(NSUB):
        pltpu.async_copy(x_hbm.at[ta.at[s]], rows_v, sem).wait()
        pltpu.sync_copy(rows_v, xs_hbm.at[qa.at[s]])


def _dispatch(ef, x, counts):
    mesh = plsc.VectorSubcoreMesh(core_axis_name="c", subcore_axis_name="s")
    run = pl.kernel(
        _dispatch_body,
        mesh=mesh,
        out_type=[
            jax.ShapeDtypeStruct((PP, DM), jnp.float32),
            jax.ShapeDtypeStruct((NWRK, NSUB, SUB), jnp.int32),
        ],
        scratch_types=[
            pltpu.VMEM((CHP,), jnp.int32),
            pltpu.VMEM((NWRK, 16), jnp.int32),
            pltpu.VMEM((16,), jnp.int32),
            pltpu.VMEM((NSUB, SUB), jnp.int32),
            pltpu.VMEM((NSUB, SUB), jnp.int32),
            pltpu.VMEM((SUB, DM), jnp.float32),
            pltpu.SemaphoreType.DMA,
        ],
        compiler_params=pltpu.CompilerParams(needs_layout_passes=False),
    )
    return run(ef, x, counts)


def _gmm_body(e_ref, xs_ref, w_ref, b_ref, ys_ref):
    y = jnp.dot(xs_ref[...], w_ref[0], preferred_element_type=jnp.float32)
    ys_ref[...] = (y + b_ref[0]) * jnp.float32(0.5)


def _gmm(e_of_tile, xs, W, b3):
    grid_spec = pltpu.PrefetchScalarGridSpec(
        num_scalar_prefetch=1,
        grid=(TMAX,),
        in_specs=[
            pl.BlockSpec((BMG, DM), lambda t, e_ref: (t, 0)),
            pl.BlockSpec((1, DM, DM), lambda t, e_ref: (e_ref[t], 0, 0)),
            pl.BlockSpec((1, 1, DM), lambda t, e_ref: (e_ref[t], 0, 0)),
        ],
        out_specs=pl.BlockSpec((BMG, DM), lambda t, e_ref: (t, 0)),
    )
    return pl.pallas_call(
        _gmm_body,
        grid_spec=grid_spec,
        out_shape=jax.ShapeDtypeStruct((PP, DM), jnp.float32),
    )(e_of_tile, xs, W, b3)


def _combine_body(ys_hbm, inv_hbm, out_hbm, inv_v, rows_v, ov, sem):
    wid = lax.axis_index("s") * 2 + lax.axis_index("c")
    lane = _lane16()
    pltpu.sync_copy(inv_hbm.at[wid], inv_v)

    def chunk_body(s, _):
        pltpu.async_copy(ys_hbm.at[inv_v.at[s]], rows_v, sem).wait()

        @plsc.parallel_loop(0, CSUB * (DM // 16), unroll=8)
        def add_body(i):
            j = i >> 6
            c = i & 63
            col = c * 16 + lane
            ra = jnp.zeros((16,), jnp.int32) + 2 * j
            a = plsc.load_gather(rows_v, [ra, col])
            bb = plsc.load_gather(rows_v, [ra + 1, col])
            ov[pl.ds(j * DM + c * 16, 16)] = a + bb

        pltpu.sync_copy(
            ov, out_hbm.at[pl.ds((wid * CHT + s * CSUB) * DM, CSUB * DM)])
        return 0

    lax.fori_loop(0, NCSUB, chunk_body, 0)


def _combine(ys, inv):
    mesh = plsc.VectorSubcoreMesh(core_axis_name="c", subcore_axis_name="s")
    run = pl.kernel(
        _combine_body,
        mesh=mesh,
        out_type=jax.ShapeDtypeStruct((NTOK * DM,), jnp.float32),
        scratch_types=[
            pltpu.VMEM((NCSUB, 2 * CSUB), jnp.int32),
            pltpu.VMEM((2 * CSUB, DM), jnp.float32),
            pltpu.VMEM((CSUB * DM,), jnp.float32),
            pltpu.SemaphoreType.DMA,
        ],
        compiler_params=pltpu.CompilerParams(needs_layout_passes=False),
    )
    return run(ys, inv)


@jax.jit
def kernel(x, W, b, Wr, br):
    xbf = x.astype(jnp.bfloat16)
    wr = jnp.pad(Wr.astype(jnp.bfloat16), ((0, 0), (0, EP - NE)))
    idx, cnt, e_of_tile = _router(xbf, wr)

    counts = cnt.reshape(NWRK, EP)[:, :16]
    ef = idx.reshape(NWRK, CHP)

    xs, inv = _dispatch(ef, x, counts)
    ys = _gmm(e_of_tile.reshape(EP), xs, W, b[:, None, :])
    out = _combine(ys, inv)
    return out.reshape(NTOK, DM)


# speedup vs baseline: 1.5910x; 1.0618x over previous
import functools

import jax
import jax.numpy as jnp
from jax import lax
from jax.experimental import pallas as pl
from jax.experimental.pallas import tpu as pltpu
from jax.experimental.pallas import tpu_sc as plsc

NE = 10
EP = 128
DM = 1024
NTOK = 8192
TOPK = 2
NPAIR = NTOK * TOPK
BM = 512
NWRK = 32
CHP = NPAIR // NWRK
CHT = NTOK // NWRK
BMG = 256
TMAX = NPAIR // BMG + NE
PP = TMAX * BMG
SUB = 32
NSUB = CHP // SUB
CSUB = 16
NCSUB = CHT // CSUB


def _router_body(xbf_ref, wr_ref, idx_ref, cnt_ref, eot_ref, tot_ref):
    m = pl.program_id(0)
    logits = jax.lax.dot_general(
        xbf_ref[...], wr_ref[...], (((1,), (0,)), ((), ())),
        preferred_element_type=jnp.float32)
    lane = jax.lax.broadcasted_iota(jnp.int32, (BM, EP), 1)
    neg = jnp.float32(-jnp.inf)
    logits = jnp.where(lane < NE, logits, neg)
    m1 = jnp.max(logits, axis=1, keepdims=True)
    i1 = jnp.min(jnp.where(logits == m1, lane, EP), axis=1, keepdims=True)
    l2 = jnp.where(lane == i1, neg, logits)
    m2 = jnp.max(l2, axis=1, keepdims=True)
    i2 = jnp.min(jnp.where(l2 == m2, lane, EP), axis=1, keepdims=True)
    idx_ref[...] = jnp.concatenate([i1, i2], axis=1)
    sel = (lane == i1) | (lane == i2)
    selc = sel.astype(jnp.int32)
    h0 = jnp.sum(selc[: BM // 2], axis=0, keepdims=True)
    h1 = jnp.sum(selc[BM // 2 :], axis=0, keepdims=True)
    cnt_ref[0] = jnp.concatenate([h0, h1], axis=0)

    @pl.when(m == 0)
    def _tot_init():
        tot_ref[...] = h0 + h1

    @pl.when(m > 0)
    def _tot_acc():
        tot_ref[...] += h0 + h1

    @pl.when(m == NTOK // BM - 1)
    def _eot():
        tot = tot_ref[...]
        tiles = jnp.where(lane[:1] < NE, (tot + (BMG - 1)) >> 8, 0)
        eot = jnp.zeros((1, EP), jnp.int32)
        run = jnp.zeros((1, 1), jnp.int32)
        for e in range(NE - 1):
            run = run + tiles[0:1, e:e + 1]
            eot += (lane[:1] >= run).astype(jnp.int32)
        eot_ref[...] = jnp.minimum(eot, NE - 1)


def _router(xbf, wr):
    return pl.pallas_call(
        _router_body,
        grid=(NTOK // BM,),
        in_specs=[
            pl.BlockSpec((BM, DM), lambda m: (m, 0)),
            pl.BlockSpec((DM, EP), lambda m: (0, 0)),
        ],
        out_specs=[
            pl.BlockSpec((BM, TOPK), lambda m: (m, 0)),
            pl.BlockSpec((1, 2, EP), lambda m: (m, 0, 0)),
            pl.BlockSpec((1, EP), lambda m: (0, 0)),
        ],
        out_shape=[
            jax.ShapeDtypeStruct((NTOK, TOPK), jnp.int32),
            jax.ShapeDtypeStruct((NTOK // BM, 2, EP), jnp.int32),
            jax.ShapeDtypeStruct((1, EP), jnp.int32),
        ],
        scratch_shapes=[pltpu.VMEM((1, EP), jnp.int32)],
    )(xbf, wr)


def _lane16():
    return jax.lax.iota(jnp.int32, 16)


def _dispatch_body(ef_hbm, x_hbm, cnt_hbm, xs_hbm, inv_hbm,
                   ef_v, all_v, roff_v, qa, ta, rows_v, rows_w, sem, sem2):
    wid = lax.axis_index("s") * 2 + lax.axis_index("c")
    lane = _lane16()
    pltpu.sync_copy(ef_hbm.at[wid], ef_v)
    pltpu.sync_copy(cnt_hbm, all_v)

    def acc_body(w, carry):
        tot, base = carry
        row = all_v[w]
        tot = tot + row
        base = base + jnp.where(w < wid, row, 0)
        return tot, base

    tot, base_w = lax.fori_loop(
        0, NWRK, acc_body,
        (jnp.zeros((16,), jnp.int32), jnp.zeros((16,), jnp.int32)))

    tiles = (tot + (BMG - 1)) >> 8
    incl = plsc.cumsum(tiles)
    pad_start = (incl - tiles) * BMG
    roff_v[...] = pad_start + base_w

    for s in range(NSUB):
        def place_body(i, _):
            k = s * (SUB // 16) + i
            v = ef_v[pl.ds(k * 16, 16)]
            base = plsc.load_gather(roff_v, [v])
            r = jnp.zeros((16,), jnp.int32)
            cnt = jnp.zeros((16,), jnp.int32)
            for e in range(NE):
                m = v == e
                mc = plsc.cumsum(m.astype(jnp.int32))
                r = jnp.where(m, mc - 1, r)
                c = plsc.all_reduce_population_count(m)
                cnt = cnt + jnp.where(lane == e, c, 0)
            q = base + r
            qa.at[s][pl.ds(i * 16, 16)] = q
            tok = (wid * CHP + k * 16 + lane) >> 1
            ta.at[s][pl.ds(i * 16, 16)] = tok
            roff_v[...] = roff_v[...] + cnt
            return 0

        lax.fori_loop(0, SUB // 16, place_body, 0)

    pltpu.sync_copy(qa, inv_hbm.at[wid])

    bufs = (rows_v, rows_w)
    gh = [None, None]
    sh = [None, None]
    gh[0] = pltpu.async_copy(x_hbm.at[ta.at[0]], bufs[0], sem)
    for s in range(NSUB):
        p = s % 2
        gh[p].wait()
        sh[p] = pltpu.async_copy(bufs[p], xs_hbm.at[qa.at[s]], sem2)
        if s + 1 < NSUB:
            q = (s + 1) % 2
            if sh[q] is not None:
                sh[q].wait()
            gh[q] = pltpu.async_copy(x_hbm.at[ta.at[s + 1]], bufs[q], sem)
    sh[(NSUB - 1) % 2].wait()
    sh[(NSUB - 2) % 2].wait()


def _dispatch(ef, x, counts):
    mesh = plsc.VectorSubcoreMesh(core_axis_name="c", subcore_axis_name="s")
    run = pl.kernel(
        _dispatch_body,
        mesh=mesh,
        out_type=[
            jax.ShapeDtypeStruct((PP, DM), jnp.float32),
            jax.ShapeDtypeStruct((NWRK, NSUB, SUB), jnp.int32),
        ],
        scratch_types=[
            pltpu.VMEM((CHP,), jnp.int32),
            pltpu.VMEM((NWRK, 16), jnp.int32),
            pltpu.VMEM((16,), jnp.int32),
            pltpu.VMEM((NSUB, SUB), jnp.int32),
            pltpu.VMEM((NSUB, SUB), jnp.int32),
            pltpu.VMEM((SUB, DM), jnp.float32),
            pltpu.VMEM((SUB, DM), jnp.float32),
            pltpu.SemaphoreType.DMA,
            pltpu.SemaphoreType.DMA,
        ],
        compiler_params=pltpu.CompilerParams(needs_layout_passes=False),
    )
    return run(ef, x, counts)


def _gmm_body(e_ref, xs_ref, w_ref, b_ref, ys_ref):
    y = jnp.dot(xs_ref[...], w_ref[0], preferred_element_type=jnp.float32)
    ys_ref[...] = (y + b_ref[0]) * jnp.float32(0.5)


def _gmm(e_of_tile, xs, W, b3):
    grid_spec = pltpu.PrefetchScalarGridSpec(
        num_scalar_prefetch=1,
        grid=(TMAX,),
        in_specs=[
            pl.BlockSpec((BMG, DM), lambda t, e_ref: (t, 0)),
            pl.BlockSpec((1, DM, DM), lambda t, e_ref: (e_ref[t], 0, 0)),
            pl.BlockSpec((1, 1, DM), lambda t, e_ref: (e_ref[t], 0, 0)),
        ],
        out_specs=pl.BlockSpec((BMG, DM), lambda t, e_ref: (t, 0)),
    )
    return pl.pallas_call(
        _gmm_body,
        grid_spec=grid_spec,
        out_shape=jax.ShapeDtypeStruct((PP, DM), jnp.float32),
    )(e_of_tile, xs, W, b3)


def _combine_body(ys_hbm, inv_hbm, out_hbm, inv_v, rows_v, rows_w, ov, sem):
    wid = lax.axis_index("s") * 2 + lax.axis_index("c")
    lane = _lane16()
    pltpu.sync_copy(inv_hbm.at[wid], inv_v)

    bufs = (rows_v, rows_w)
    gh = [None, None]
    gh[0] = pltpu.async_copy(ys_hbm.at[inv_v.at[0]], bufs[0], sem)
    for s in range(NCSUB):
        p = s % 2
        gh[p].wait()
        if s + 1 < NCSUB:
            gh[(s + 1) % 2] = pltpu.async_copy(
                ys_hbm.at[inv_v.at[s + 1]], bufs[(s + 1) % 2], sem)
        buf = bufs[p]

        @plsc.parallel_loop(0, CSUB * (DM // 16), unroll=8)
        def add_body(i):
            j = i >> 6
            c = i & 63
            col = c * 16 + lane
            ra = jnp.zeros((16,), jnp.int32) + 2 * j
            a = plsc.load_gather(buf, [ra, col])
            bb = plsc.load_gather(buf, [ra + 1, col])
            ov[pl.ds(j * DM + c * 16, 16)] = a + bb

        pltpu.sync_copy(
            ov, out_hbm.at[pl.ds((wid * CHT + s * CSUB) * DM, CSUB * DM)])


def _combine(ys, inv):
    mesh = plsc.VectorSubcoreMesh(core_axis_name="c", subcore_axis_name="s")
    run = pl.kernel(
        _combine_body,
        mesh=mesh,
        out_type=jax.ShapeDtypeStruct((NTOK * DM,), jnp.float32),
        scratch_types=[
            pltpu.VMEM((NCSUB, 2 * CSUB), jnp.int32),
            pltpu.VMEM((2 * CSUB, DM), jnp.float32),
            pltpu.VMEM((2 * CSUB, DM), jnp.float32),
            pltpu.VMEM((CSUB * DM,), jnp.float32),
            pltpu.SemaphoreType.DMA,
        ],
        compiler_params=pltpu.CompilerParams(needs_layout_passes=False),
    )
    return run(ys, inv)


@jax.jit
def kernel(x, W, b, Wr, br):
    xbf = x.astype(jnp.bfloat16)
    wr = jnp.pad(Wr.astype(jnp.bfloat16), ((0, 0), (0, EP - NE)))
    idx, cnt, e_of_tile = _router(xbf, wr)

    counts = cnt.reshape(NWRK, EP)[:, :16]
    ef = idx.reshape(NWRK, CHP)

    xs, inv = _dispatch(ef, x, counts)
    ys = _gmm(e_of_tile.reshape(EP), xs, W, b[:, None, :])
    out = _combine(ys, inv)
    return out.reshape(NTOK, DM)


# speedup vs baseline: 1.6794x; 1.0555x over previous
import functools

import jax
import jax.numpy as jnp
from jax import lax
from jax.experimental import pallas as pl
from jax.experimental.pallas import tpu as pltpu
from jax.experimental.pallas import tpu_sc as plsc

NE = 10
EP = 128
DM = 1024
NTOK = 8192
TOPK = 2
NPAIR = NTOK * TOPK
BM = 512
NWRK = 32
CHP = NPAIR // NWRK
CHT = NTOK // NWRK
BMG = 256
TMAX = NPAIR // BMG + NE
PP = TMAX * BMG
SUB = 32
NSUB = CHP // SUB
CSUB = 16
NCSUB = CHT // CSUB


def _router_body(x_ref, wr_ref, idx_ref, cnt_ref, eot_ref, tot_ref):
    m = pl.program_id(0)
    logits = jax.lax.dot_general(
        x_ref[...].astype(jnp.bfloat16), wr_ref[...], (((1,), (0,)), ((), ())),
        preferred_element_type=jnp.float32)
    lane = jax.lax.broadcasted_iota(jnp.int32, (BM, EP), 1)
    neg = jnp.float32(-jnp.inf)
    logits = jnp.where(lane < NE, logits, neg)
    m1 = jnp.max(logits, axis=1, keepdims=True)
    i1 = jnp.min(jnp.where(logits == m1, lane, EP), axis=1, keepdims=True)
    l2 = jnp.where(lane == i1, neg, logits)
    m2 = jnp.max(l2, axis=1, keepdims=True)
    i2 = jnp.min(jnp.where(l2 == m2, lane, EP), axis=1, keepdims=True)
    idx_ref[...] = jnp.concatenate([i1, i2], axis=1)
    sel = (lane == i1) | (lane == i2)
    selc = sel.astype(jnp.int32)
    h0 = jnp.sum(selc[: BM // 2], axis=0, keepdims=True)
    h1 = jnp.sum(selc[BM // 2 :], axis=0, keepdims=True)
    cnt_ref[0] = jnp.concatenate([h0, h1], axis=0)

    @pl.when(m == 0)
    def _tot_init():
        tot_ref[...] = h0 + h1

    @pl.when(m > 0)
    def _tot_acc():
        tot_ref[...] += h0 + h1

    @pl.when(m == NTOK // BM - 1)
    def _eot():
        tot = tot_ref[...]
        tiles = jnp.where(lane[:1] < NE, (tot + (BMG - 1)) >> 8, 0)
        eot = jnp.zeros((1, EP), jnp.int32)
        run = jnp.zeros((1, 1), jnp.int32)
        for e in range(NE - 1):
            run = run + tiles[0:1, e:e + 1]
            eot += (lane[:1] >= run).astype(jnp.int32)
        eot_ref[...] = jnp.minimum(eot, NE - 1)


def _router(x, wr):
    return pl.pallas_call(
        _router_body,
        grid=(NTOK // BM,),
        in_specs=[
            pl.BlockSpec((BM, DM), lambda m: (m, 0)),
            pl.BlockSpec((DM, EP), lambda m: (0, 0)),
        ],
        out_specs=[
            pl.BlockSpec((BM, TOPK), lambda m: (m, 0)),
            pl.BlockSpec((1, 2, EP), lambda m: (m, 0, 0)),
            pl.BlockSpec((1, EP), lambda m: (0, 0)),
        ],
        out_shape=[
            jax.ShapeDtypeStruct((NTOK, TOPK), jnp.int32),
            jax.ShapeDtypeStruct((NTOK // BM, 2, EP), jnp.int32),
            jax.ShapeDtypeStruct((1, EP), jnp.int32),
        ],
        scratch_shapes=[pltpu.VMEM((1, EP), jnp.int32)],
    )(x, wr)


def _lane16():
    return jax.lax.iota(jnp.int32, 16)


def _dispatch_body(ef_hbm, x_hbm, cnt_hbm, xs_hbm, inv_hbm,
                   ef_v, all_v, roff_v, qa, ta, rows_v, rows_w, sem, sem2):
    wid = lax.axis_index("s") * 2 + lax.axis_index("c")
    lane = _lane16()
    pltpu.sync_copy(ef_hbm.at[wid], ef_v)
    pltpu.sync_copy(cnt_hbm, all_v)

    def acc_body(w, carry):
        tot, base = carry
        row = all_v[w]
        tot = tot + row
        base = base + jnp.where(w < wid, row, 0)
        return tot, base

    tot, base_w = lax.fori_loop(
        0, NWRK, acc_body,
        (jnp.zeros((16,), jnp.int32), jnp.zeros((16,), jnp.int32)))

    tiles = (tot + (BMG - 1)) >> 8
    incl = plsc.cumsum(tiles)
    pad_start = (incl - tiles) * BMG
    roff_v[...] = pad_start + base_w

    for s in range(NSUB):
        def place_body(i, _):
            k = s * (SUB // 16) + i
            v = ef_v[pl.ds(k * 16, 16)]
            base = plsc.load_gather(roff_v, [v])
            r = jnp.zeros((16,), jnp.int32)
            cnt = jnp.zeros((16,), jnp.int32)
            for e in range(NE):
                m = v == e
                mc = plsc.cumsum(m.astype(jnp.int32))
                r = jnp.where(m, mc - 1, r)
                c = plsc.all_reduce_population_count(m)
                cnt = cnt + jnp.where(lane == e, c, 0)
            q = base + r
            qa.at[s][pl.ds(i * 16, 16)] = q
            tok = (wid * CHP + k * 16 + lane) >> 1
            ta.at[s][pl.ds(i * 16, 16)] = tok
            roff_v[...] = roff_v[...] + cnt
            return 0

        lax.fori_loop(0, SUB // 16, place_body, 0)

    pltpu.sync_copy(qa, inv_hbm.at[wid])

    bufs = (rows_v, rows_w)
    gh = [None, None]
    sh = [None, None]
    gh[0] = pltpu.async_copy(x_hbm.at[ta.at[0]], bufs[0], sem)
    for s in range(NSUB):
        p = s % 2
        gh[p].wait()
        sh[p] = pltpu.async_copy(bufs[p], xs_hbm.at[qa.at[s]], sem2)
        if s + 1 < NSUB:
            q = (s + 1) % 2
            if sh[q] is not None:
                sh[q].wait()
            gh[q] = pltpu.async_copy(x_hbm.at[ta.at[s + 1]], bufs[q], sem)
    sh[(NSUB - 1) % 2].wait()
    sh[(NSUB - 2) % 2].wait()


def _dispatch(ef, x, counts):
    mesh = plsc.VectorSubcoreMesh(core_axis_name="c", subcore_axis_name="s")
    run = pl.kernel(
        _dispatch_body,
        mesh=mesh,
        out_type=[
            jax.ShapeDtypeStruct((PP, DM), jnp.float32),
            jax.ShapeDtypeStruct((NWRK, NSUB, SUB), jnp.int32),
        ],
        scratch_types=[
            pltpu.VMEM((CHP,), jnp.int32),
            pltpu.VMEM((NWRK, 16), jnp.int32),
            pltpu.VMEM((16,), jnp.int32),
            pltpu.VMEM((NSUB, SUB), jnp.int32),
            pltpu.VMEM((NSUB, SUB), jnp.int32),
            pltpu.VMEM((SUB, DM), jnp.float32),
            pltpu.VMEM((SUB, DM), jnp.float32),
            pltpu.SemaphoreType.DMA,
            pltpu.SemaphoreType.DMA,
        ],
        compiler_params=pltpu.CompilerParams(needs_layout_passes=False),
    )
    return run(ef, x, counts)


def _gmm_body(e_ref, xs_ref, w_ref, b_ref, ys_ref):
    y = jnp.dot(xs_ref[...], w_ref[0], preferred_element_type=jnp.float32)
    ys_ref[...] = (y + b_ref[0]) * jnp.float32(0.5)


def _gmm(e_of_tile, xs, W, b3):
    grid_spec = pltpu.PrefetchScalarGridSpec(
        num_scalar_prefetch=1,
        grid=(TMAX,),
        in_specs=[
            pl.BlockSpec((BMG, DM), lambda t, e_ref: (t, 0)),
            pl.BlockSpec((1, DM, DM), lambda t, e_ref: (e_ref[t], 0, 0)),
            pl.BlockSpec((1, 1, DM), lambda t, e_ref: (e_ref[t], 0, 0)),
        ],
        out_specs=pl.BlockSpec((BMG, DM), lambda t, e_ref: (t, 0)),
    )
    return pl.pallas_call(
        _gmm_body,
        grid_spec=grid_spec,
        out_shape=jax.ShapeDtypeStruct((PP, DM), jnp.float32),
    )(e_of_tile, xs, W, b3)


def _combine_body(ys_hbm, inv_hbm, out_hbm, inv_v, rows_v, rows_w, ov, sem):
    wid = lax.axis_index("s") * 2 + lax.axis_index("c")
    lane = _lane16()
    pltpu.sync_copy(inv_hbm.at[wid], inv_v)

    bufs = (rows_v, rows_w)
    gh = [None, None]
    gh[0] = pltpu.async_copy(ys_hbm.at[inv_v.at[0]], bufs[0], sem)
    for s in range(NCSUB):
        p = s % 2
        gh[p].wait()
        if s + 1 < NCSUB:
            gh[(s + 1) % 2] = pltpu.async_copy(
                ys_hbm.at[inv_v.at[s + 1]], bufs[(s + 1) % 2], sem)
        buf = bufs[p]

        @plsc.parallel_loop(0, CSUB * (DM // 16), unroll=8)
        def add_body(i):
            j = i >> 6
            c = i & 63
            col = c * 16 + lane
            ra = jnp.zeros((16,), jnp.int32) + 2 * j
            a = plsc.load_gather(buf, [ra, col])
            bb = plsc.load_gather(buf, [ra + 1, col])
            ov[pl.ds(j * DM + c * 16, 16)] = a + bb

        pltpu.sync_copy(
            ov, out_hbm.at[pl.ds((wid * CHT + s * CSUB) * DM, CSUB * DM)])


def _combine(ys, inv):
    mesh = plsc.VectorSubcoreMesh(core_axis_name="c", subcore_axis_name="s")
    run = pl.kernel(
        _combine_body,
        mesh=mesh,
        out_type=jax.ShapeDtypeStruct((NTOK * DM,), jnp.float32),
        scratch_types=[
            pltpu.VMEM((NCSUB, 2 * CSUB), jnp.int32),
            pltpu.VMEM((2 * CSUB, DM), jnp.float32),
            pltpu.VMEM((2 * CSUB, DM), jnp.float32),
            pltpu.VMEM((CSUB * DM,), jnp.float32),
            pltpu.SemaphoreType.DMA,
        ],
        compiler_params=pltpu.CompilerParams(needs_layout_passes=False),
    )
    return run(ys, inv)


@jax.jit
def kernel(x, W, b, Wr, br):
    wr = jnp.pad(Wr.astype(jnp.bfloat16), ((0, 0), (0, EP - NE)))
    idx, cnt, e_of_tile = _router(x, wr)

    counts = cnt.reshape(NWRK, EP)[:, :16]
    ef = idx.reshape(NWRK, CHP)

    xs, inv = _dispatch(ef, x, counts)
    ys = _gmm(e_of_tile.reshape(EP), xs, W, b[:, None, :])
    out = _combine(ys, inv)
    return out.reshape(NTOK, DM)


# speedup vs baseline: 1.7330x; 1.0319x over previous
import functools

import jax
import jax.numpy as jnp
from jax import lax
from jax.experimental import pallas as pl
from jax.experimental.pallas import tpu as pltpu
from jax.experimental.pallas import tpu_sc as plsc

NE = 10
EP = 128
DM = 1024
NTOK = 8192
TOPK = 2
NPAIR = NTOK * TOPK
BM = 512
NWRK = 32
CHP = NPAIR // NWRK
CHT = NTOK // NWRK
BMG = 512
BMG_SHIFT = 9
TMAX = NPAIR // BMG + NE
PP = TMAX * BMG
SUB = 32
NSUB = CHP // SUB
CSUB = 16
NCSUB = CHT // CSUB


def _router_body(x_ref, wr_ref, idx_ref, cnt_ref, eot_ref, tot_ref):
    m = pl.program_id(0)
    logits = jax.lax.dot_general(
        x_ref[...].astype(jnp.bfloat16), wr_ref[...], (((1,), (0,)), ((), ())),
        preferred_element_type=jnp.float32)
    lane = jax.lax.broadcasted_iota(jnp.int32, (BM, EP), 1)
    neg = jnp.float32(-jnp.inf)
    logits = jnp.where(lane < NE, logits, neg)
    m1 = jnp.max(logits, axis=1, keepdims=True)
    i1 = jnp.min(jnp.where(logits == m1, lane, EP), axis=1, keepdims=True)
    l2 = jnp.where(lane == i1, neg, logits)
    m2 = jnp.max(l2, axis=1, keepdims=True)
    i2 = jnp.min(jnp.where(l2 == m2, lane, EP), axis=1, keepdims=True)
    idx_ref[...] = jnp.concatenate([i1, i2], axis=1)
    sel = (lane == i1) | (lane == i2)
    selc = sel.astype(jnp.int32)
    h0 = jnp.sum(selc[: BM // 2], axis=0, keepdims=True)
    h1 = jnp.sum(selc[BM // 2 :], axis=0, keepdims=True)
    cnt_ref[0] = jnp.concatenate([h0, h1], axis=0)

    @pl.when(m == 0)
    def _tot_init():
        tot_ref[...] = h0 + h1

    @pl.when(m > 0)
    def _tot_acc():
        tot_ref[...] += h0 + h1

    @pl.when(m == NTOK // BM - 1)
    def _eot():
        tot = tot_ref[...]
        tiles = jnp.where(lane[:1] < NE, (tot + (BMG - 1)) >> BMG_SHIFT, 0)
        eot = jnp.zeros((1, EP), jnp.int32)
        run = jnp.zeros((1, 1), jnp.int32)
        for e in range(NE - 1):
            run = run + tiles[0:1, e:e + 1]
            eot += (lane[:1] >= run).astype(jnp.int32)
        eot_ref[...] = jnp.minimum(eot, NE - 1)


def _router(x, wr):
    return pl.pallas_call(
        _router_body,
        grid=(NTOK // BM,),
        in_specs=[
            pl.BlockSpec((BM, DM), lambda m: (m, 0)),
            pl.BlockSpec((DM, EP), lambda m: (0, 0)),
        ],
        out_specs=[
            pl.BlockSpec((BM, TOPK), lambda m: (m, 0)),
            pl.BlockSpec((1, 2, EP), lambda m: (m, 0, 0)),
            pl.BlockSpec((1, EP), lambda m: (0, 0)),
        ],
        out_shape=[
            jax.ShapeDtypeStruct((NTOK, TOPK), jnp.int32),
            jax.ShapeDtypeStruct((NTOK // BM, 2, EP), jnp.int32),
            jax.ShapeDtypeStruct((1, EP), jnp.int32),
        ],
        scratch_shapes=[pltpu.VMEM((1, EP), jnp.int32)],
    )(x, wr)


def _lane16():
    return jax.lax.iota(jnp.int32, 16)


def _dispatch_body(ef_hbm, x_hbm, cnt_hbm, xs_hbm, inv_hbm,
                   ef_v, all_v, roff_v, qa, ta, rows_v, rows_w, sem, sem2):
    wid = lax.axis_index("s") * 2 + lax.axis_index("c")
    lane = _lane16()
    pltpu.sync_copy(ef_hbm.at[wid], ef_v)
    pltpu.sync_copy(cnt_hbm, all_v)

    def acc_body(w, carry):
        tot, base = carry
        row = all_v[w]
        tot = tot + row
        base = base + jnp.where(w < wid, row, 0)
        return tot, base

    tot, base_w = lax.fori_loop(
        0, NWRK, acc_body,
        (jnp.zeros((16,), jnp.int32), jnp.zeros((16,), jnp.int32)))

    tiles = (tot + (BMG - 1)) >> BMG_SHIFT
    incl = plsc.cumsum(tiles)
    pad_start = (incl - tiles) * BMG
    roff_v[...] = pad_start + base_w

    for s in range(NSUB):
        def place_body(i, _):
            k = s * (SUB // 16) + i
            v = ef_v[pl.ds(k * 16, 16)]
            base = plsc.load_gather(roff_v, [v])
            r = jnp.zeros((16,), jnp.int32)
            cnt = jnp.zeros((16,), jnp.int32)
            for e in range(NE):
                m = v == e
                mc = plsc.cumsum(m.astype(jnp.int32))
                r = jnp.where(m, mc - 1, r)
                c = plsc.all_reduce_population_count(m)
                cnt = cnt + jnp.where(lane == e, c, 0)
            q = base + r
            qa.at[s][pl.ds(i * 16, 16)] = q
            tok = (wid * CHP + k * 16 + lane) >> 1
            ta.at[s][pl.ds(i * 16, 16)] = tok
            roff_v[...] = roff_v[...] + cnt
            return 0

        lax.fori_loop(0, SUB // 16, place_body, 0)

    pltpu.sync_copy(qa, inv_hbm.at[wid])

    bufs = (rows_v, rows_w)
    gh = [None, None]
    sh = [None, None]
    gh[0] = pltpu.async_copy(x_hbm.at[ta.at[0]], bufs[0], sem)
    for s in range(NSUB):
        p = s % 2
        gh[p].wait()
        sh[p] = pltpu.async_copy(bufs[p], xs_hbm.at[qa.at[s]], sem2)
        if s + 1 < NSUB:
            q = (s + 1) % 2
            if sh[q] is not None:
                sh[q].wait()
            gh[q] = pltpu.async_copy(x_hbm.at[ta.at[s + 1]], bufs[q], sem)
    sh[(NSUB - 1) % 2].wait()
    sh[(NSUB - 2) % 2].wait()


def _dispatch(ef, x, counts):
    mesh = plsc.VectorSubcoreMesh(core_axis_name="c", subcore_axis_name="s")
    run = pl.kernel(
        _dispatch_body,
        mesh=mesh,
        out_type=[
            jax.ShapeDtypeStruct((PP, DM), jnp.float32),
            jax.ShapeDtypeStruct((NWRK, NSUB, SUB), jnp.int32),
        ],
        scratch_types=[
            pltpu.VMEM((CHP,), jnp.int32),
            pltpu.VMEM((NWRK, 16), jnp.int32),
            pltpu.VMEM((16,), jnp.int32),
            pltpu.VMEM((NSUB, SUB), jnp.int32),
            pltpu.VMEM((NSUB, SUB), jnp.int32),
            pltpu.VMEM((SUB, DM), jnp.float32),
            pltpu.VMEM((SUB, DM), jnp.float32),
            pltpu.SemaphoreType.DMA,
            pltpu.SemaphoreType.DMA,
        ],
        compiler_params=pltpu.CompilerParams(needs_layout_passes=False),
    )
    return run(ef, x, counts)


def _gmm_body(e_ref, xs_ref, w_ref, b_ref, ys_ref):
    y = jnp.dot(xs_ref[...], w_ref[0], preferred_element_type=jnp.float32)
    ys_ref[...] = (y + b_ref[0]) * jnp.float32(0.5)


def _gmm(e_of_tile, xs, W, b3):
    grid_spec = pltpu.PrefetchScalarGridSpec(
        num_scalar_prefetch=1,
        grid=(TMAX,),
        in_specs=[
            pl.BlockSpec((BMG, DM), lambda t, e_ref: (t, 0)),
            pl.BlockSpec((1, DM, DM), lambda t, e_ref: (e_ref[t], 0, 0)),
            pl.BlockSpec((1, 1, DM), lambda t, e_ref: (e_ref[t], 0, 0)),
        ],
        out_specs=pl.BlockSpec((BMG, DM), lambda t, e_ref: (t, 0)),
    )
    return pl.pallas_call(
        _gmm_body,
        grid_spec=grid_spec,
        out_shape=jax.ShapeDtypeStruct((PP, DM), jnp.float32),
    )(e_of_tile, xs, W, b3)


def _combine_body(ys_hbm, inv_hbm, out_hbm, inv_v, rows_v, rows_w, ov, sem):
    wid = lax.axis_index("s") * 2 + lax.axis_index("c")
    lane = _lane16()
    pltpu.sync_copy(inv_hbm.at[wid], inv_v)

    bufs = (rows_v, rows_w)
    gh = [None, None]
    gh[0] = pltpu.async_copy(ys_hbm.at[inv_v.at[0]], bufs[0], sem)
    for s in range(NCSUB):
        p = s % 2
        gh[p].wait()
        if s + 1 < NCSUB:
            gh[(s + 1) % 2] = pltpu.async_copy(
                ys_hbm.at[inv_v.at[s + 1]], bufs[(s + 1) % 2], sem)
        buf = bufs[p]

        @plsc.parallel_loop(0, CSUB * (DM // 16), unroll=8)
        def add_body(i):
            j = i >> 6
            c = i & 63
            col = c * 16 + lane
            ra = jnp.zeros((16,), jnp.int32) + 2 * j
            a = plsc.load_gather(buf, [ra, col])
            bb = plsc.load_gather(buf, [ra + 1, col])
            ov[pl.ds(j * DM + c * 16, 16)] = a + bb

        pltpu.sync_copy(
            ov, out_hbm.at[pl.ds((wid * CHT + s * CSUB) * DM, CSUB * DM)])


def _combine(ys, inv):
    mesh = plsc.VectorSubcoreMesh(core_axis_name="c", subcore_axis_name="s")
    run = pl.kernel(
        _combine_body,
        mesh=mesh,
        out_type=jax.ShapeDtypeStruct((NTOK * DM,), jnp.float32),
        scratch_types=[
            pltpu.VMEM((NCSUB, 2 * CSUB), jnp.int32),
            pltpu.VMEM((2 * CSUB, DM), jnp.float32),
            pltpu.VMEM((2 * CSUB, DM), jnp.float32),
            pltpu.VMEM((CSUB * DM,), jnp.float32),
            pltpu.SemaphoreType.DMA,
        ],
        compiler_params=pltpu.CompilerParams(needs_layout_passes=False),
    )
    return run(ys, inv)


@jax.jit
def kernel(x, W, b, Wr, br):
    wr = jnp.pad(Wr.astype(jnp.bfloat16), ((0, 0), (0, EP - NE)))
    idx, cnt, e_of_tile = _router(x, wr)

    counts = cnt.reshape(NWRK, EP)[:, :16]
    ef = idx.reshape(NWRK, CHP)

    xs, inv = _dispatch(ef, x, counts)
    ys = _gmm(e_of_tile.reshape(EP), xs, W, b[:, None, :])
    out = _combine(ys, inv)
    return out.reshape(NTOK, DM)
